# baseline (speedup 1.0000x reference)
import jax
import jax.numpy as jnp
from jax.experimental import pallas as pl
from jax.experimental.pallas import tpu as pltpu

_N = 10000
_D = 128
_CHUNK = 2000
_NCHUNK = _N // _CHUNK


def _mlp_chunk(x, w1, w2):
    h = jax.lax.dot_general(
        x.astype(jnp.bfloat16), w1, (((1,), (1,)), ((), ())),
        preferred_element_type=jnp.float32,
    )
    h = jnp.maximum(h, 0.0).astype(jnp.bfloat16)
    return jax.lax.dot_general(
        h, w2, (((1,), (1,)), ((), ())),
        preferred_element_type=jnp.float32,
    )


def _pipelined_kernel(x_hbm, w_emb_ref, w_out_ref, o_hbm,
                      x_buf, o_buf, in_sem, out_sem):
    def in_copy(i):
        return pltpu.make_async_copy(
            x_hbm.at[pl.ds(i * _CHUNK, _CHUNK), :],
            x_buf.at[i % 2],
            in_sem.at[i % 2],
        )

    def out_copy(i):
        return pltpu.make_async_copy(
            o_buf.at[i % 2],
            o_hbm.at[pl.ds(i * _CHUNK, _CHUNK), :],
            out_sem.at[i % 2],
        )

    w1 = w_emb_ref[...].astype(jnp.bfloat16)
    w2 = w_out_ref[...].astype(jnp.bfloat16)

    in_copy(0).start()
    for i in range(_NCHUNK):
        if i + 1 < _NCHUNK:
            in_copy(i + 1).start()
        in_copy(i).wait()
        y = _mlp_chunk(x_buf[i % 2], w1, w2)
        if i >= 2:
            out_copy(i - 2).wait()
        o_buf[i % 2] = y
        out_copy(i).start()
    out_copy(_NCHUNK - 2).wait()
    out_copy(_NCHUNK - 1).wait()


def kernel(features, edge_index, W_emb, W_out, betas):
    del edge_index, betas
    return pl.pallas_call(
        _pipelined_kernel,
        in_specs=[
            pl.BlockSpec(memory_space=pltpu.MemorySpace.HBM),
            pl.BlockSpec(memory_space=pltpu.MemorySpace.VMEM),
            pl.BlockSpec(memory_space=pltpu.MemorySpace.VMEM),
        ],
        out_specs=pl.BlockSpec(memory_space=pltpu.MemorySpace.HBM),
        out_shape=jax.ShapeDtypeStruct((_N, _D), jnp.float32),
        scratch_shapes=[
            pltpu.VMEM((2, _CHUNK, _D), jnp.float32),
            pltpu.VMEM((2, _CHUNK, _D), jnp.float32),
            pltpu.SemaphoreType.DMA((2,)),
            pltpu.SemaphoreType.DMA((2,)),
        ],
    )(features, W_emb, W_out)
